# baseline (speedup 1.0000x reference)
import functools

import jax
import jax.numpy as jnp
from jax import lax
from jax.experimental import pallas as pl
from jax.experimental.pallas import tpu as pltpu
from jax.experimental.pallas import tpu_sc as plsc

NUM_FREQ = 4
IN_DIM = 4
OUT_DIM = 4
NUM_BINS = 34
ROW = OUT_DIM * IN_DIM * NUM_FREQ
PROW = 2 * ROW
E = 800000
P = E // 2

NC = 2
NS = 16
NW = NC * NS
L = 16

PAIRS_W = P // NW
CH = 128
NB = 4
BLK = NB * CH
NFULL = PAIRS_W // BLK
REM = PAIRS_W - NFULL * BLK
REM_SIZES = [CH] * (REM // CH) + ([REM % CH] if REM % CH else [])

_MAGIC = 8388608.0


def _bins_from_dists(d):
    x = jnp.clip((d - 2.4) / 0.4, 0.0, 33.0)
    r = (x + _MAGIC) - _MAGIC
    return r.astype(jnp.int32)


_mesh = plsc.VectorSubcoreMesh(core_axis_name="c", subcore_axis_name="s")


@functools.partial(
    pl.kernel,
    mesh=_mesh,
    out_type=jax.ShapeDtypeStruct((P, PROW), jnp.float32),
    scratch_types=[
        pltpu.VMEM_SHARED((NUM_BINS * NUM_BINS, PROW), jnp.float32),
        pltpu.VMEM((2 * BLK,), jnp.float32),
        pltpu.VMEM((2 * BLK,), jnp.int32),
        pltpu.VMEM((BLK,), jnp.int32),
        [pltpu.VMEM((CH, PROW), jnp.float32) for _ in range(NB)],
        pltpu.SemaphoreType.DMA,
        pltpu.SemaphoreType.DMA,
    ],
    compiler_params=pltpu.CompilerParams(use_tc_tiling_on_sc=False,
                                         needs_layout_passes=False),
)
def _radial_sc(dists_hbm, ptable_hbm, out_hbm, tbl_s, d_v, bin_v, idx_v, rows,
               sem_g, sem_w):
    sid = lax.axis_index("s")
    wid = sid * NC + lax.axis_index("c")
    pbase_w = wid * PAIRS_W

    @pl.when(sid == 0)
    def _():
        pltpu.sync_copy(ptable_hbm, tbl_s)

    plsc.subcore_barrier()

    lane = lax.iota(jnp.int32, L)

    def compute_indices(n_pairs):
        for k in range(-(-2 * n_pairs // L)):
            bin_v[pl.ds(k * L, L)] = _bins_from_dists(d_v[pl.ds(k * L, L)])
        for k in range(-(-n_pairs // L)):
            even = plsc.load_gather(bin_v, [lane * 2 + (2 * L * k)])
            odd = plsc.load_gather(bin_v, [lane * 2 + (2 * L * k + 1)])
            idx_v[pl.ds(k * L, L)] = even * NUM_BINS + odd

    def fire_block(pbase, sizes):
        pltpu.sync_copy(dists_hbm.at[pl.ds(2 * pbase, 2 * sum(sizes))],
                        d_v.at[pl.ds(0, 2 * sum(sizes))])
        compute_indices(sum(sizes))
        copies, off = [], 0
        for b, sz in enumerate(sizes):
            copies.append(pltpu.async_copy(
                ptable_hbm.at[idx_v.at[pl.ds(off, sz)]],
                rows[b].at[pl.ds(0, sz)], sem_g))
            off += sz
        for c in copies:
            c.wait()
        off = 0
        for b, sz in enumerate(sizes):
            pltpu.async_copy(rows[b].at[pl.ds(0, sz)],
                             out_hbm.at[pl.ds(pbase + off, sz)], sem_w)
            off += sz

    def drain_writes(sizes):
        for b, sz in enumerate(sizes):
            pltpu.make_async_copy(rows[b].at[pl.ds(0, sz)],
                                  out_hbm.at[pl.ds(pbase_w, sz)], sem_w).wait()

    @pl.loop(0, NFULL)
    def _(g):
        @pl.when(g > 0)
        def _():
            drain_writes([CH] * NB)

        fire_block(pbase_w + g * BLK, [CH] * NB)

    drain_writes([CH] * NB)
    fire_block(pbase_w + NFULL * BLK, REM_SIZES)
    drain_writes(REM_SIZES)


def kernel(dists, bin_embedding):
    left = jnp.broadcast_to(bin_embedding[:, None, :],
                            (NUM_BINS, NUM_BINS, ROW))
    right = jnp.broadcast_to(bin_embedding[None, :, :],
                             (NUM_BINS, NUM_BINS, ROW))
    ptable = jnp.concatenate([left, right], -1).reshape(NUM_BINS * NUM_BINS,
                                                        PROW)
    flat = _radial_sc(dists.reshape(E), ptable)
    return flat.reshape(E, OUT_DIM, 1, IN_DIM, 1, NUM_FREQ)


# speedup vs baseline: 6.6131x; 6.6131x over previous
import functools

import jax
import jax.numpy as jnp
from jax import lax
from jax.experimental import pallas as pl
from jax.experimental.pallas import tpu as pltpu
from jax.experimental.pallas import tpu_sc as plsc

NUM_FREQ = 4
IN_DIM = 4
OUT_DIM = 4
NUM_BINS = 34
ROW = OUT_DIM * IN_DIM * NUM_FREQ
E = 800000
ETILES = E // 128

NC = 2
NS = 16
NW = NC * NS
L = 16

ST = 5
EPB = ST * 128
NSUP = ETILES // ST
NIT = -(-NSUP // NW)

_MAGIC = 8388608.0


def _bins_from_dists(d):
    x = jnp.clip((d - 2.4) / 0.4, 0.0, 33.0)
    r = (x + _MAGIC) - _MAGIC
    return r.astype(jnp.int32)


_mesh = plsc.VectorSubcoreMesh(core_axis_name="c", subcore_axis_name="s")


@functools.partial(
    pl.kernel,
    mesh=_mesh,
    out_type=jax.ShapeDtypeStruct((ROW // NUM_FREQ, ETILES, NUM_FREQ, 128),
                                  jnp.float32),
    scratch_types=[
        pltpu.VMEM((NUM_BINS * ROW,), jnp.float32),
        [pltpu.VMEM((EPB,), jnp.float32) for _ in range(2)],
        [pltpu.VMEM((EPB,), jnp.int32) for _ in range(2)],
        [pltpu.VMEM((ROW // NUM_FREQ, ST, NUM_FREQ, 128), jnp.float32)
         for _ in range(2)],
        [pltpu.SemaphoreType.DMA for _ in range(2)],
    ],
    compiler_params=pltpu.CompilerParams(use_tc_tiling_on_sc=False,
                                         needs_layout_passes=False),
)
def _radial_sc(dists_hbm, table_hbm, out_hbm, tbl_v, d_v, bin_v, tbuf, sem_w):
    wid = lax.axis_index("s") * NC + lax.axis_index("c")

    pltpu.sync_copy(table_hbm, tbl_v)

    def drain(b):
        pltpu.make_async_copy(
            tbuf[b], out_hbm.at[:, pl.ds(0, ST)], sem_w[b]).wait()

    def process(s, b):
        base = s * EPB
        pltpu.sync_copy(dists_hbm.at[pl.ds(base, EPB)], d_v[b])

        @pl.loop(0, EPB // L)
        def _(g):
            bin_v[b][pl.ds(g * L, L)] = _bins_from_dists(d_v[b][pl.ds(g * L, L)])

        @pl.loop(0, ST)
        def _(t):
            @pl.loop(0, 128 // L)
            def _(eg):
                idx = bin_v[b][pl.ds(t * 128 + eg * L, L)] * ROW
                for c in range(ROW):
                    v = plsc.load_gather(tbl_v, [idx])
                    tbuf[b][c // NUM_FREQ, t, c % NUM_FREQ,
                            pl.ds(eg * L, L)] = v
                    if c < ROW - 1:
                        idx = idx + 1

        for oi in range(ROW // NUM_FREQ):
            pltpu.async_copy(tbuf[b].at[oi],
                             out_hbm.at[oi, pl.ds(s * ST, ST)], sem_w[b])

    @pl.loop(0, NIT, step=2)
    def _(j):
        for b in range(2):
            k = j + b
            s = wid + k * NW

            @pl.when(k >= 2)
            def _():
                drain(b)

            @pl.when(s < NSUP)
            def _():
                process(s, b)

    drain((NIT - 2) % 2)

    @pl.when(wid < NSUP - (NIT - 1) * NW)
    def _():
        drain((NIT - 1) % 2)


def kernel(dists, bin_embedding):
    x = _radial_sc(dists.reshape(E), bin_embedding.reshape(NUM_BINS * ROW))
    x = x.reshape(OUT_DIM, IN_DIM, ETILES, NUM_FREQ, 128)
    x = x.transpose(2, 4, 0, 1, 3).reshape(E, OUT_DIM, IN_DIM, NUM_FREQ)
    return x[:, :, None, :, None, :]


# speedup vs baseline: 12.5377x; 1.8959x over previous
import functools

import jax
import jax.numpy as jnp
from jax import lax
from jax.experimental import pallas as pl
from jax.experimental.pallas import tpu as pltpu
from jax.experimental.pallas import tpu_sc as plsc

NUM_FREQ = 4
IN_DIM = 4
OUT_DIM = 4
NUM_BINS = 34
ROW = OUT_DIM * IN_DIM * NUM_FREQ
E = 800000
ETILES = E // 128

NC = 2
NS = 16
NW = NC * NS
L = 16

ST = 5
EPB = ST * 128
NSUP = ETILES // ST
NIT = -(-NSUP // NW)

_MAGIC = 8388608.0


def _bins_from_dists(d):
    x = jnp.clip((d - 2.4) / 0.4, 0.0, 33.0)
    r = (x + _MAGIC) - _MAGIC
    return r.astype(jnp.int32)


_mesh = plsc.VectorSubcoreMesh(core_axis_name="c", subcore_axis_name="s")


@functools.partial(
    pl.kernel,
    mesh=_mesh,
    out_type=jax.ShapeDtypeStruct((ROW // NUM_FREQ, ETILES, NUM_FREQ, 128),
                                  jnp.float32),
    scratch_types=[
        pltpu.VMEM((NUM_BINS * (ROW + 1),), jnp.float32),
        [pltpu.VMEM((EPB,), jnp.float32) for _ in range(2)],
        [pltpu.VMEM((EPB,), jnp.int32) for _ in range(2)],
        [pltpu.VMEM((ROW // NUM_FREQ, ST, NUM_FREQ, 128), jnp.float32)
         for _ in range(2)],
        [pltpu.SemaphoreType.DMA for _ in range(2)],
    ],
    compiler_params=pltpu.CompilerParams(use_tc_tiling_on_sc=False,
                                         needs_layout_passes=False),
)
def _radial_sc(dists_hbm, table_hbm, out_hbm, tbl_v, d_v, bin_v, tbuf, sem_w):
    wid = lax.axis_index("s") * NC + lax.axis_index("c")

    pltpu.sync_copy(table_hbm, tbl_v)

    def drain(b):
        pltpu.make_async_copy(
            tbuf[b], out_hbm.at[:, pl.ds(0, ST)], sem_w[b]).wait()

    def process(s, b):
        base = s * EPB
        pltpu.sync_copy(dists_hbm.at[pl.ds(base, EPB)], d_v[b])

        @pl.loop(0, EPB // L)
        def _(g):
            bin_v[b][pl.ds(g * L, L)] = _bins_from_dists(d_v[b][pl.ds(g * L, L)])

        @pl.loop(0, ST)
        def _(t):
            @pl.loop(0, 128 // L)
            def _(eg):
                idx = bin_v[b][pl.ds(t * 128 + eg * L, L)] * (ROW + 1)
                for c in range(ROW):
                    v = plsc.load_gather(tbl_v, [idx])
                    tbuf[b][c // NUM_FREQ, t, c % NUM_FREQ,
                            pl.ds(eg * L, L)] = v
                    if c < ROW - 1:
                        idx = idx + 1

        for oi in range(ROW // NUM_FREQ):
            pltpu.async_copy(tbuf[b].at[oi],
                             out_hbm.at[oi, pl.ds(s * ST, ST)], sem_w[b])

    @pl.loop(0, NIT, step=2)
    def _(j):
        for b in range(2):
            k = j + b
            s = wid + k * NW

            @pl.when(k >= 2)
            def _():
                drain(b)

            @pl.when(s < NSUP)
            def _():
                process(s, b)

    drain((NIT - 2) % 2)

    @pl.when(wid < NSUP - (NIT - 1) * NW)
    def _():
        drain((NIT - 1) % 2)


def kernel(dists, bin_embedding):
    tpad = jnp.pad(bin_embedding, ((0, 0), (0, 1))).reshape(NUM_BINS * (ROW + 1))
    x = _radial_sc(dists.reshape(E), tpad)
    x = x.reshape(OUT_DIM, IN_DIM, ETILES, NUM_FREQ, 128)
    x = x.transpose(2, 4, 0, 1, 3).reshape(E, OUT_DIM, IN_DIM, NUM_FREQ)
    return x[:, :, None, :, None, :]
